# baseline (speedup 1.0000x reference)
import functools

import jax
import jax.numpy as jnp
from jax import lax
from jax.experimental import pallas as pl
from jax.experimental.pallas import tpu as pltpu
from jax.experimental.pallas import tpu_sc as plsc

_B = 16384
_F = 26
_V = 100000
_D = 32
_DENSE = 13

_NC = 2
_NS = 16
_NW = _NC * _NS

_NR = (_B * _F) // 128
_RPW = _NR // _NW

_VC = 512
_NVB = 196
_RPF = _NVB * (_VC // 4)
_FM_ROWS = (_B * _F * _D) // 128


def _tc_pack_table(tab_t):
    def body(x_ref, o_ref):
        x = x_ref[0]
        o_ref[...] = x.reshape(_D, _VC // 4, 4).transpose(1, 2, 0).reshape(
            _VC // 4, 128)

    return pl.pallas_call(
        body,
        grid=(_F, _NVB),
        in_specs=[pl.BlockSpec((1, _D, _VC), lambda f, v: (f, 0, v))],
        out_specs=pl.BlockSpec((_VC // 4, 128), lambda f, v: (f * _NVB + v, 0)),
        out_shape=jax.ShapeDtypeStruct((_F * _RPF, 128), jnp.float32),
    )(tab_t)


def _sc_gather(didx2, lidx2, tw, ltab):
    mesh = plsc.VectorSubcoreMesh(
        core_axis_name="c", subcore_axis_name="s",
        num_cores=_NC, num_subcores=_NS)

    @functools.partial(
        pl.kernel,
        out_type=(
            jax.ShapeDtypeStruct((_FM_ROWS, 128), jnp.float32),
            jax.ShapeDtypeStruct((_NR, 128), jnp.float32),
        ),
        mesh=mesh,
        compiler_params=pltpu.CompilerParams(needs_layout_passes=False),
        scratch_types=[
            pltpu.VMEM((_RPW, 128), jnp.int32),
            pltpu.VMEM((_RPW, 128), jnp.int32),
            pltpu.VMEM((_RPW, 128), jnp.int32),
            pltpu.VMEM((_RPW, 128), jnp.float32),
            pltpu.VMEM((128, 128), jnp.float32),
            pltpu.VMEM((128, 128), jnp.float32),
            pltpu.VMEM((32, 128), jnp.float32),
            pltpu.VMEM((32, 128), jnp.float32),
            pltpu.SemaphoreType.DMA,
            pltpu.SemaphoreType.DMA,
            pltpu.SemaphoreType.DMA,
        ],
    )
    def body(didx_hbm, lidx_hbm, tw_hbm, ltab_hbm, fm_out, lin_out,
             m_v, q_v, lidx_v, lval_v, wide0, wide1, ext0, ext1,
             sem_a, sem_b, sem_l):
        wid = lax.axis_index("s") * _NC + lax.axis_index("c")
        row0 = wid * _RPW

        pltpu.sync_copy(didx_hbm.at[pl.ds(row0, _RPW)], m_v)
        pltpu.sync_copy(lidx_hbm.at[pl.ds(row0, _RPW)], lidx_v)

        iota = lax.iota(jnp.int32, 16)

        def prep(j, carry):
            for t in range(8):
                sl = pl.ds(t * 16, 16)
                f = ((j * 128 + t * 16) + iota) % _F
                raw = m_v[j, sl]
                m_v[j, sl] = f * _RPF + jax.lax.shift_right_logical(raw, 2)
                q_v[j, sl] = jax.lax.bitwise_and(raw, 3)
                lidx_v[j, sl] = lidx_v[j, sl] + f * _V
            return carry
        lax.fori_loop(0, _RPW, prep, 0)

        def lin_group(g, carry):
            for t in range(8):
                r = g * 8 + t
                pltpu.async_copy(ltab_hbm.at[lidx_v.at[r]], lval_v.at[r], sem_l)
            pltpu.make_async_copy(
                lin_out.at[pl.ds(row0, 8)],
                lval_v.at[pl.ds(g * 8, 8)], sem_l).wait()
            return carry
        lax.fori_loop(0, 13, lin_group, 0)
        pltpu.sync_copy(lval_v, lin_out.at[pl.ds(row0, _RPW)])

        def extract(j, wide, ext):
            jb = jnp.broadcast_to(j, (16,))

            def egrp(g0, carry):
                for u in range(4):
                    g = g0 * 4 + u
                    w = g * 16 + iota
                    i = jax.lax.shift_right_logical(w, 5)
                    q = plsc.load_gather(q_v, (jb, i))
                    col = q * 32 + jax.lax.bitwise_and(w, 31)
                    vals = plsc.load_gather(wide, (i, col))
                    ext[g // 8, pl.ds((g % 8) * 16, 16)] = vals
                return carry
            lax.fori_loop(0, 64, egrp, 0)

        pltpu.async_copy(tw_hbm.at[m_v.at[0]], wide0, sem_a)

        def dnn_body(k, carry):
            j0 = 2 * k
            j1 = j0 + 1
            pltpu.async_copy(tw_hbm.at[m_v.at[j1]], wide1, sem_b)
            pltpu.make_async_copy(fm_out.at[pl.ds(0, 128)], wide0, sem_a).wait()
            extract(j0, wide0, ext0)
            pltpu.sync_copy(ext0, fm_out.at[pl.ds((row0 + j0) * 32, 32)])

            @pl.when(k < _RPW // 2 - 1)
            def _():
                pltpu.async_copy(tw_hbm.at[m_v.at[j0 + 2]], wide0, sem_a)

            pltpu.make_async_copy(fm_out.at[pl.ds(0, 128)], wide1, sem_b).wait()
            extract(j1, wide1, ext1)
            pltpu.sync_copy(ext1, fm_out.at[pl.ds((row0 + j1) * 32, 32)])
            return carry
        lax.fori_loop(0, _RPW // 2, dnn_body, 0)

    return body(didx2, lidx2, tw, ltab)


def _tc_head(fm2, lval, dnn_dense, lin_dense,
             w1s, w1d, b1, g1, be1, w2, b2, g2, be2, w3, b3, lin_w, lin_b):
    bb = 512
    grid = (_B // bb,)

    def body(fm_ref, lv_ref, dd_ref, ld_ref,
             w1s_ref, w1d_ref, b1_ref, g1_ref, be1_ref,
             w2_ref, b2_ref, g2_ref, be2_ref,
             w3_ref, b3_ref, linw_ref, linb_ref, out_ref):
        fm = fm_ref[...]
        r = lax.broadcasted_iota(jnp.int32, (_F * _D, _D), 0)
        c = lax.broadcasted_iota(jnp.int32, (_F * _D, _D), 1)
        s = (r % _D == c).astype(jnp.float32)
        dn = (((1,), (1,)), ((), ()))
        mm = lambda x, w: lax.dot_general(
            x, w, dimension_numbers=dn, preferred_element_type=jnp.float32)
        sum_e = lax.dot_general(fm, s, dimension_numbers=(((1,), (0,)), ((), ())),
                                preferred_element_type=jnp.float32)
        ssq = lax.dot_general(fm * fm, s, dimension_numbers=(((1,), (0,)), ((), ())),
                              preferred_element_type=jnp.float32)
        cross = 0.5 * jnp.sum(sum_e * sum_e - ssq, axis=1, keepdims=True)

        h = mm(fm, w1s_ref[...]) + mm(dd_ref[...], w1d_ref[...]) + b1_ref[...]
        h = jnp.maximum(h, 0.0)
        m = jnp.mean(h, axis=1, keepdims=True)
        xc = h - m
        v = jnp.mean(xc * xc, axis=1, keepdims=True)
        h = xc * lax.rsqrt(v + 1e-5) * g1_ref[...] + be1_ref[...]

        h = jnp.maximum(mm(h, w2_ref[...]) + b2_ref[...], 0.0)
        m = jnp.mean(h, axis=1, keepdims=True)
        xc = h - m
        v = jnp.mean(xc * xc, axis=1, keepdims=True)
        h = xc * lax.rsqrt(v + 1e-5) * g2_ref[...] + be2_ref[...]

        dnn_logit = jnp.maximum(
            jnp.sum(h * w3_ref[...], axis=1, keepdims=True) + b3_ref[0, 0], 0.0)

        lin_logit = (jnp.sum(ld_ref[...] * linw_ref[...], axis=1, keepdims=True)
                     + linb_ref[0, 0]
                     + jnp.sum(lv_ref[...], axis=1, keepdims=True))
        out_ref[...] = lin_logit + dnn_logit + cross

    full = lambda shape: pl.BlockSpec(shape, lambda i: (0, 0))
    return pl.pallas_call(
        body,
        grid=grid,
        in_specs=[
            pl.BlockSpec((bb, _F * _D), lambda i: (i, 0)),
            pl.BlockSpec((bb, _F), lambda i: (i, 0)),
            pl.BlockSpec((bb, _DENSE), lambda i: (i, 0)),
            pl.BlockSpec((bb, _DENSE), lambda i: (i, 0)),
            full((128, _F * _D)), full((128, _DENSE)),
            full((1, 128)), full((1, 128)), full((1, 128)),
            full((64, 128)), full((1, 64)), full((1, 64)), full((1, 64)),
            full((1, 64)), full((1, 1)), full((1, _DENSE)), full((1, 1)),
        ],
        out_specs=pl.BlockSpec((bb, 1), lambda i: (i, 0)),
        out_shape=jax.ShapeDtypeStruct((_B, 1), jnp.float32),
    )(fm2, lval, dnn_dense, lin_dense,
      w1s, w1d, b1, g1, be1, w2, b2, g2, be2, w3, b3, lin_w, lin_b)


def kernel(linear_dense_data, dnn_dense_data, linear_tables, dnn_tables,
           lin_W, lin_b, W1, b1, ln1_g, ln1_b, W2, b2, ln2_g, ln2_b, W3, b3,
           linear_sparse_data, dnn_sparse_data):
    didx2 = dnn_sparse_data.astype(jnp.int32).reshape(_NR, 128)
    lidx2 = linear_sparse_data.astype(jnp.int32).reshape(_NR, 128)
    tab_t = jnp.transpose(dnn_tables, (0, 2, 1))
    tw = _tc_pack_table(tab_t)
    ltab = linear_tables.reshape(_F * _V)

    fm_rows, lin_rows = _sc_gather(didx2, lidx2, tw, ltab)
    fm2 = fm_rows.reshape(_B, _F * _D)
    lval = lin_rows.reshape(_B, _F)

    w1d = W1[:, :_DENSE]
    w1s = W1[:, _DENSE:]
    out = _tc_head(
        fm2, lval, dnn_dense_data, linear_dense_data,
        w1s, w1d,
        b1.reshape(1, 128), ln1_g.reshape(1, 128), ln1_b.reshape(1, 128),
        W2, b2.reshape(1, 64), ln2_g.reshape(1, 64), ln2_b.reshape(1, 64),
        W3.reshape(1, 64), b3.reshape(1, 1),
        lin_W.reshape(1, _DENSE), lin_b.reshape(1, 1))
    return out


# speedup vs baseline: 8.4894x; 8.4894x over previous
import functools

import jax
import jax.numpy as jnp
from jax import lax
from jax.experimental import pallas as pl
from jax.experimental.pallas import tpu as pltpu
from jax.experimental.pallas import tpu_sc as plsc

_B = 16384
_F = 26
_V = 100000
_D = 32
_DENSE = 13

_NC = 2
_NS = 16
_NW = _NC * _NS

_NR = (_B * _F) // 128
_RPW = _NR // _NW

_RPF = 25088
_UC = 6272
_NUB = _RPF // _UC
_FM_ROWS = (_B * _F * _D) // 128


def _tc_pack_table(tab_t):
    def body(x0, x1, x2, x3, o_ref):
        o_ref[...] = jnp.concatenate(
            [x0[0], x1[0], x2[0], x3[0]], axis=0).T

    def imap(k):
        return lambda f, u: (f, 0, k * _NUB + u)

    return pl.pallas_call(
        body,
        grid=(_F, _NUB),
        in_specs=[pl.BlockSpec((1, _D, _UC), imap(k)) for k in range(4)],
        out_specs=pl.BlockSpec((_UC, 128), lambda f, u: (f * _NUB + u, 0)),
        out_shape=jax.ShapeDtypeStruct((_F * _RPF, 128), jnp.float32),
    )(tab_t, tab_t, tab_t, tab_t)


def _sc_gather(didx2, lidx2, tw, ltab):
    mesh = plsc.VectorSubcoreMesh(
        core_axis_name="c", subcore_axis_name="s",
        num_cores=_NC, num_subcores=_NS)

    @functools.partial(
        pl.kernel,
        out_type=(
            jax.ShapeDtypeStruct((_FM_ROWS, 128), jnp.float32),
            jax.ShapeDtypeStruct((_NR, 128), jnp.float32),
        ),
        mesh=mesh,
        compiler_params=pltpu.CompilerParams(needs_layout_passes=False),
        scratch_types=[
            pltpu.VMEM((_RPW, 128), jnp.int32),
            pltpu.VMEM((_RPW, 128), jnp.int32),
            pltpu.VMEM((_RPW, 128), jnp.int32),
            pltpu.VMEM((_RPW, 128), jnp.float32),
            pltpu.VMEM((128, 128), jnp.float32),
            pltpu.VMEM((128, 128), jnp.float32),
            pltpu.VMEM((32, 128), jnp.float32),
            pltpu.VMEM((32, 128), jnp.float32),
            pltpu.SemaphoreType.DMA,
            pltpu.SemaphoreType.DMA,
            pltpu.SemaphoreType.DMA,
        ],
    )
    def body(didx_hbm, lidx_hbm, tw_hbm, ltab_hbm, fm_out, lin_out,
             m_v, q_v, lidx_v, lval_v, wide0, wide1, ext0, ext1,
             sem_a, sem_b, sem_l):
        wid = lax.axis_index("s") * _NC + lax.axis_index("c")
        row0 = wid * _RPW

        pltpu.sync_copy(didx_hbm.at[pl.ds(row0, _RPW)], m_v)
        pltpu.sync_copy(lidx_hbm.at[pl.ds(row0, _RPW)], lidx_v)

        iota = lax.iota(jnp.int32, 16)

        def prep(j, carry):
            for t in range(8):
                sl = pl.ds(t * 16, 16)
                f = ((j * 128 + t * 16) + iota) % _F
                raw = m_v[j, sl]
                m_v[j, sl] = f * _RPF + jax.lax.rem(raw, _RPF)
                q_v[j, sl] = jax.lax.div(raw, _RPF)
                lidx_v[j, sl] = lidx_v[j, sl] + f * _V
            return carry
        lax.fori_loop(0, _RPW, prep, 0)

        def lin_group(g, carry):
            for t in range(8):
                r = g * 8 + t
                pltpu.async_copy(ltab_hbm.at[lidx_v.at[r]], lval_v.at[r], sem_l)
            pltpu.make_async_copy(
                lin_out.at[pl.ds(row0, 8)],
                lval_v.at[pl.ds(g * 8, 8)], sem_l).wait()
            return carry
        lax.fori_loop(0, 13, lin_group, 0)
        pltpu.sync_copy(lval_v, lin_out.at[pl.ds(row0, _RPW)])

        def extract(j, wide, ext):
            jb = jnp.broadcast_to(j, (16,))

            def egrp(g0, carry):
                for u in range(4):
                    g = g0 * 4 + u
                    w = g * 16 + iota
                    i = jax.lax.shift_right_logical(w, 5)
                    q = plsc.load_gather(q_v, (jb, i))
                    col = q * 32 + jax.lax.bitwise_and(w, 31)
                    vals = plsc.load_gather(wide, (i, col))
                    ext[g // 8, pl.ds((g % 8) * 16, 16)] = vals
                return carry
            lax.fori_loop(0, 64, egrp, 0)

        pltpu.async_copy(tw_hbm.at[m_v.at[0]], wide0, sem_a)

        def dnn_body(k, carry):
            j0 = 2 * k
            j1 = j0 + 1
            pltpu.async_copy(tw_hbm.at[m_v.at[j1]], wide1, sem_b)
            pltpu.make_async_copy(fm_out.at[pl.ds(0, 128)], wide0, sem_a).wait()
            extract(j0, wide0, ext0)
            pltpu.sync_copy(ext0, fm_out.at[pl.ds((row0 + j0) * 32, 32)])

            @pl.when(k < _RPW // 2 - 1)
            def _():
                pltpu.async_copy(tw_hbm.at[m_v.at[j0 + 2]], wide0, sem_a)

            pltpu.make_async_copy(fm_out.at[pl.ds(0, 128)], wide1, sem_b).wait()
            extract(j1, wide1, ext1)
            pltpu.sync_copy(ext1, fm_out.at[pl.ds((row0 + j1) * 32, 32)])
            return carry
        lax.fori_loop(0, _RPW // 2, dnn_body, 0)

    return body(didx2, lidx2, tw, ltab)


def _tc_head(fm2, lval, dnn_dense, lin_dense,
             w1s, w1d, b1, g1, be1, w2, b2, g2, be2, w3, b3, lin_w, lin_b):
    bb = 512
    grid = (_B // bb,)

    def body(fm_ref, lv_ref, dd_ref, ld_ref,
             w1s_ref, w1d_ref, b1_ref, g1_ref, be1_ref,
             w2_ref, b2_ref, g2_ref, be2_ref,
             w3_ref, b3_ref, linw_ref, linb_ref, out_ref):
        fm = fm_ref[...]
        r = lax.broadcasted_iota(jnp.int32, (_F * _D, _D), 0)
        c = lax.broadcasted_iota(jnp.int32, (_F * _D, _D), 1)
        s = (r % _D == c).astype(jnp.float32)
        dn = (((1,), (1,)), ((), ()))
        mm = lambda x, w: lax.dot_general(
            x, w, dimension_numbers=dn, preferred_element_type=jnp.float32)
        sum_e = lax.dot_general(fm, s, dimension_numbers=(((1,), (0,)), ((), ())),
                                preferred_element_type=jnp.float32)
        ssq = lax.dot_general(fm * fm, s, dimension_numbers=(((1,), (0,)), ((), ())),
                              preferred_element_type=jnp.float32)
        cross = 0.5 * jnp.sum(sum_e * sum_e - ssq, axis=1, keepdims=True)

        h = mm(fm, w1s_ref[...]) + mm(dd_ref[...], w1d_ref[...]) + b1_ref[...]
        h = jnp.maximum(h, 0.0)
        m = jnp.mean(h, axis=1, keepdims=True)
        xc = h - m
        v = jnp.mean(xc * xc, axis=1, keepdims=True)
        h = xc * lax.rsqrt(v + 1e-5) * g1_ref[...] + be1_ref[...]

        h = jnp.maximum(mm(h, w2_ref[...]) + b2_ref[...], 0.0)
        m = jnp.mean(h, axis=1, keepdims=True)
        xc = h - m
        v = jnp.mean(xc * xc, axis=1, keepdims=True)
        h = xc * lax.rsqrt(v + 1e-5) * g2_ref[...] + be2_ref[...]

        dnn_logit = jnp.maximum(
            jnp.sum(h * w3_ref[...], axis=1, keepdims=True) + b3_ref[0, 0], 0.0)

        lin_logit = (jnp.sum(ld_ref[...] * linw_ref[...], axis=1, keepdims=True)
                     + linb_ref[0, 0]
                     + jnp.sum(lv_ref[...], axis=1, keepdims=True))
        out_ref[...] = lin_logit + dnn_logit + cross

    full = lambda shape: pl.BlockSpec(shape, lambda i: (0, 0))
    return pl.pallas_call(
        body,
        grid=grid,
        in_specs=[
            pl.BlockSpec((bb, _F * _D), lambda i: (i, 0)),
            pl.BlockSpec((bb, _F), lambda i: (i, 0)),
            pl.BlockSpec((bb, _DENSE), lambda i: (i, 0)),
            pl.BlockSpec((bb, _DENSE), lambda i: (i, 0)),
            full((128, _F * _D)), full((128, _DENSE)),
            full((1, 128)), full((1, 128)), full((1, 128)),
            full((64, 128)), full((1, 64)), full((1, 64)), full((1, 64)),
            full((1, 64)), full((1, 1)), full((1, _DENSE)), full((1, 1)),
        ],
        out_specs=pl.BlockSpec((bb, 1), lambda i: (i, 0)),
        out_shape=jax.ShapeDtypeStruct((_B, 1), jnp.float32),
    )(fm2, lval, dnn_dense, lin_dense,
      w1s, w1d, b1, g1, be1, w2, b2, g2, be2, w3, b3, lin_w, lin_b)


def kernel(linear_dense_data, dnn_dense_data, linear_tables, dnn_tables,
           lin_W, lin_b, W1, b1, ln1_g, ln1_b, W2, b2, ln2_g, ln2_b, W3, b3,
           linear_sparse_data, dnn_sparse_data):
    didx2 = dnn_sparse_data.astype(jnp.int32).reshape(_NR, 128)
    lidx2 = linear_sparse_data.astype(jnp.int32).reshape(_NR, 128)
    tab_t = jnp.transpose(dnn_tables, (0, 2, 1))
    tw = _tc_pack_table(tab_t)
    ltab = linear_tables.reshape(_F * _V)

    fm_rows, lin_rows = _sc_gather(didx2, lidx2, tw, ltab)
    fm2 = fm_rows.reshape(_B, _F * _D)
    lval = lin_rows.reshape(_B, _F)

    w1d = W1[:, :_DENSE]
    w1s = W1[:, _DENSE:]
    out = _tc_head(
        fm2, lval, dnn_dense_data, linear_dense_data,
        w1s, w1d,
        b1.reshape(1, 128), ln1_g.reshape(1, 128), ln1_b.reshape(1, 128),
        W2, b2.reshape(1, 64), ln2_g.reshape(1, 64), ln2_b.reshape(1, 64),
        W3.reshape(1, 64), b3.reshape(1, 1),
        lin_W.reshape(1, _DENSE), lin_b.reshape(1, 1))
    return out


# speedup vs baseline: 9.1014x; 1.0721x over previous
import functools

import jax
import jax.numpy as jnp
from jax import lax
from jax.experimental import pallas as pl
from jax.experimental.pallas import tpu as pltpu
from jax.experimental.pallas import tpu_sc as plsc

_B = 16384
_F = 26
_V = 100000
_D = 32
_DENSE = 13

_NC = 2
_NS = 16
_NW = _NC * _NS

_NR = (_B * _F) // 128
_RPW = _NR // _NW

_RPF = 25088
_UC = 6272
_NUB = _RPF // _UC
_FM_ROWS = (_B * _F * _D) // 128


def _tc_pack_table(tab_t):
    def body(x0, x1, x2, x3, o_ref):
        o_ref[...] = jnp.concatenate(
            [x0[0], x1[0], x2[0], x3[0]], axis=0).T

    def imap(k):
        return lambda f, u: (f, 0, k * _NUB + u)

    return pl.pallas_call(
        body,
        grid=(_F, _NUB),
        in_specs=[pl.BlockSpec((1, _D, _UC), imap(k)) for k in range(4)],
        out_specs=pl.BlockSpec((_UC, 128), lambda f, u: (f * _NUB + u, 0)),
        out_shape=jax.ShapeDtypeStruct((_F * _RPF, 128), jnp.float32),
    )(tab_t, tab_t, tab_t, tab_t)


_SC_MESH = plsc.VectorSubcoreMesh(
    core_axis_name="c", subcore_axis_name="s",
    num_cores=_NC, num_subcores=_NS)


def _sc_prep_lin(didx2, lidx2, ltab):
    @functools.partial(
        pl.kernel,
        out_type=(
            jax.ShapeDtypeStruct((_NR, 128), jnp.float32),
            jax.ShapeDtypeStruct((_NR, 128), jnp.int32),
            jax.ShapeDtypeStruct((_NR, 128), jnp.int32),
        ),
        mesh=_SC_MESH,
        compiler_params=pltpu.CompilerParams(needs_layout_passes=False),
        scratch_types=[
            pltpu.VMEM((_RPW, 128), jnp.int32),
            pltpu.VMEM((_RPW, 128), jnp.int32),
            pltpu.VMEM((_RPW, 128), jnp.int32),
            pltpu.VMEM((_RPW, 128), jnp.float32),
            pltpu.SemaphoreType.DMA,
        ],
    )
    def body(didx_hbm, lidx_hbm, ltab_hbm, lin_out, m_out, q_out,
             m_v, q_v, lidx_v, lval_v, sem_l):
        wid = lax.axis_index("s") * _NC + lax.axis_index("c")
        row0 = wid * _RPW

        pltpu.sync_copy(didx_hbm.at[pl.ds(row0, _RPW)], m_v)
        pltpu.sync_copy(lidx_hbm.at[pl.ds(row0, _RPW)], lidx_v)

        iota = lax.iota(jnp.int32, 16)

        def prep(j, carry):
            for t in range(8):
                sl = pl.ds(t * 16, 16)
                f = ((j * 128 + t * 16) + iota) % _F
                raw = m_v[j, sl]
                m_v[j, sl] = f * _RPF + jax.lax.rem(raw, _RPF)
                q_v[j, sl] = jax.lax.div(raw, _RPF)
                lidx_v[j, sl] = lidx_v[j, sl] + f * _V
            return carry
        lax.fori_loop(0, _RPW, prep, 0)

        def lin_group(g, carry):
            for t in range(8):
                r = g * 8 + t
                pltpu.async_copy(ltab_hbm.at[lidx_v.at[r]], lval_v.at[r], sem_l)
            pltpu.make_async_copy(
                lin_out.at[pl.ds(row0, 8)],
                lval_v.at[pl.ds(g * 8, 8)], sem_l).wait()
            return carry
        lax.fori_loop(0, 13, lin_group, 0)
        pltpu.sync_copy(lval_v, lin_out.at[pl.ds(row0, _RPW)])
        pltpu.sync_copy(m_v, m_out.at[pl.ds(row0, _RPW)])
        pltpu.sync_copy(q_v, q_out.at[pl.ds(row0, _RPW)])

    return body(didx2, lidx2, ltab)


def _sc_gather(m2, q2, tw):
    @functools.partial(
        pl.kernel,
        out_type=jax.ShapeDtypeStruct((_FM_ROWS, 128), jnp.float32),
        mesh=_SC_MESH,
        compiler_params=pltpu.CompilerParams(needs_layout_passes=False),
        scratch_types=[
            pltpu.VMEM((_RPW, 128), jnp.int32),
            pltpu.VMEM((_RPW, 128), jnp.int32),
            pltpu.VMEM((128, 128), jnp.float32),
            pltpu.VMEM((128, 128), jnp.float32),
            pltpu.VMEM((128, 128), jnp.float32),
            pltpu.VMEM((128, 128), jnp.float32),
            pltpu.VMEM((32, 128), jnp.float32),
            pltpu.SemaphoreType.DMA,
            pltpu.SemaphoreType.DMA,
            pltpu.SemaphoreType.DMA,
            pltpu.SemaphoreType.DMA,
        ],
    )
    def body(m_hbm, q_hbm, tw_hbm, fm_out,
             m_v, q_v, w0, w1, w2, w3, ext, s0, s1, s2, s3):
        wid = lax.axis_index("s") * _NC + lax.axis_index("c")
        row0 = wid * _RPW
        wides = (w0, w1, w2, w3)
        sems = (s0, s1, s2, s3)

        pltpu.sync_copy(m_hbm.at[pl.ds(row0, _RPW)], m_v)
        pltpu.sync_copy(q_hbm.at[pl.ds(row0, _RPW)], q_v)

        iota = lax.iota(jnp.int32, 16)

        def extract(j, wide):
            jb = jnp.broadcast_to(j, (16,))

            def egrp(g0, carry):
                for u in range(4):
                    g = g0 * 4 + u
                    w = g * 16 + iota
                    i = jax.lax.shift_right_logical(w, 5)
                    q = plsc.load_gather(q_v, (jb, i))
                    col = q * 32 + jax.lax.bitwise_and(w, 31)
                    vals = plsc.load_gather(wide, (i, col))
                    ext[g // 8, pl.ds((g % 8) * 16, 16)] = vals
                return carry
            lax.fori_loop(0, 64, egrp, 0)

        for p in range(3):
            pltpu.async_copy(tw_hbm.at[m_v.at[p]], wides[p], sems[p])

        def dnn_body(k, carry):
            for u in range(4):
                j = 4 * k + u
                pltpu.make_async_copy(
                    fm_out.at[pl.ds(0, 128)], wides[u], sems[u]).wait()

                @pl.when(j < _RPW - 3)
                def _():
                    pltpu.async_copy(
                        tw_hbm.at[m_v.at[j + 3]], wides[(u + 3) % 4],
                        sems[(u + 3) % 4])

                extract(j, wides[u])
                pltpu.sync_copy(ext, fm_out.at[pl.ds((row0 + j) * 32, 32)])
            return carry
        lax.fori_loop(0, _RPW // 4, dnn_body, 0)

    return body(m2, q2, tw)


def _tc_head(fm2, lval, dnn_dense, lin_dense,
             w1s, w1d, b1, g1, be1, w2, b2, g2, be2, w3, b3, lin_w, lin_b):
    bb = 512
    grid = (_B // bb,)

    def body(fm_ref, lv_ref, dd_ref, ld_ref,
             w1s_ref, w1d_ref, b1_ref, g1_ref, be1_ref,
             w2_ref, b2_ref, g2_ref, be2_ref,
             w3_ref, b3_ref, linw_ref, linb_ref, out_ref):
        fm = fm_ref[...]
        r = lax.broadcasted_iota(jnp.int32, (_F * _D, _D), 0)
        c = lax.broadcasted_iota(jnp.int32, (_F * _D, _D), 1)
        s = (r % _D == c).astype(jnp.float32)
        dn = (((1,), (1,)), ((), ()))
        mm = lambda x, w: lax.dot_general(
            x, w, dimension_numbers=dn, preferred_element_type=jnp.float32)
        sum_e = lax.dot_general(fm, s, dimension_numbers=(((1,), (0,)), ((), ())),
                                preferred_element_type=jnp.float32)
        ssq = lax.dot_general(fm * fm, s, dimension_numbers=(((1,), (0,)), ((), ())),
                              preferred_element_type=jnp.float32)
        cross = 0.5 * jnp.sum(sum_e * sum_e - ssq, axis=1, keepdims=True)

        h = mm(fm, w1s_ref[...]) + mm(dd_ref[...], w1d_ref[...]) + b1_ref[...]
        h = jnp.maximum(h, 0.0)
        m = jnp.mean(h, axis=1, keepdims=True)
        xc = h - m
        v = jnp.mean(xc * xc, axis=1, keepdims=True)
        h = xc * lax.rsqrt(v + 1e-5) * g1_ref[...] + be1_ref[...]

        h = jnp.maximum(mm(h, w2_ref[...]) + b2_ref[...], 0.0)
        m = jnp.mean(h, axis=1, keepdims=True)
        xc = h - m
        v = jnp.mean(xc * xc, axis=1, keepdims=True)
        h = xc * lax.rsqrt(v + 1e-5) * g2_ref[...] + be2_ref[...]

        dnn_logit = jnp.maximum(
            jnp.sum(h * w3_ref[...], axis=1, keepdims=True) + b3_ref[0, 0], 0.0)

        lin_logit = (jnp.sum(ld_ref[...] * linw_ref[...], axis=1, keepdims=True)
                     + linb_ref[0, 0]
                     + jnp.sum(lv_ref[...], axis=1, keepdims=True))
        out_ref[...] = lin_logit + dnn_logit + cross

    full = lambda shape: pl.BlockSpec(shape, lambda i: (0, 0))
    return pl.pallas_call(
        body,
        grid=grid,
        in_specs=[
            pl.BlockSpec((bb, _F * _D), lambda i: (i, 0)),
            pl.BlockSpec((bb, _F), lambda i: (i, 0)),
            pl.BlockSpec((bb, _DENSE), lambda i: (i, 0)),
            pl.BlockSpec((bb, _DENSE), lambda i: (i, 0)),
            full((128, _F * _D)), full((128, _DENSE)),
            full((1, 128)), full((1, 128)), full((1, 128)),
            full((64, 128)), full((1, 64)), full((1, 64)), full((1, 64)),
            full((1, 64)), full((1, 1)), full((1, _DENSE)), full((1, 1)),
        ],
        out_specs=pl.BlockSpec((bb, 1), lambda i: (i, 0)),
        out_shape=jax.ShapeDtypeStruct((_B, 1), jnp.float32),
    )(fm2, lval, dnn_dense, lin_dense,
      w1s, w1d, b1, g1, be1, w2, b2, g2, be2, w3, b3, lin_w, lin_b)


def kernel(linear_dense_data, dnn_dense_data, linear_tables, dnn_tables,
           lin_W, lin_b, W1, b1, ln1_g, ln1_b, W2, b2, ln2_g, ln2_b, W3, b3,
           linear_sparse_data, dnn_sparse_data):
    didx2 = dnn_sparse_data.astype(jnp.int32).reshape(_NR, 128)
    lidx2 = linear_sparse_data.astype(jnp.int32).reshape(_NR, 128)
    tab_t = jnp.transpose(dnn_tables, (0, 2, 1))
    tw = _tc_pack_table(tab_t)
    ltab = linear_tables.reshape(_F * _V)

    lin_rows, m2, q2 = _sc_prep_lin(didx2, lidx2, ltab)
    fm_rows = _sc_gather(m2, q2, tw)
    fm2 = fm_rows.reshape(_B, _F * _D)
    lval = lin_rows.reshape(_B, _F)

    w1d = W1[:, :_DENSE]
    w1s = W1[:, _DENSE:]
    out = _tc_head(
        fm2, lval, dnn_dense_data, linear_dense_data,
        w1s, w1d,
        b1.reshape(1, 128), ln1_g.reshape(1, 128), ln1_b.reshape(1, 128),
        W2, b2.reshape(1, 64), ln2_g.reshape(1, 64), ln2_b.reshape(1, 64),
        W3.reshape(1, 64), b3.reshape(1, 1),
        lin_W.reshape(1, _DENSE), lin_b.reshape(1, 1))
    return out


# speedup vs baseline: 11.1271x; 1.2226x over previous
import functools

import jax
import jax.numpy as jnp
from jax import lax
from jax.experimental import pallas as pl
from jax.experimental.pallas import tpu as pltpu
from jax.experimental.pallas import tpu_sc as plsc

_B = 16384
_F = 26
_V = 100000
_D = 32
_DENSE = 13

_NC = 2
_NS = 16
_NW = _NC * _NS

_NR = (_B * _F) // 128
_RPW = _NR // _NW

_RPF = 25088
_UC = 6272
_NUB = _RPF // _UC
_FM_ROWS = (_B * _F * _D) // 128


def _tc_pack_table(tab_t):
    def body(x0, x1, x2, x3, o_ref):
        o_ref[...] = jnp.concatenate(
            [x0[0], x1[0], x2[0], x3[0]], axis=0).T

    def imap(k):
        return lambda f, u: (f, 0, k * _NUB + u)

    return pl.pallas_call(
        body,
        grid=(_F, _NUB),
        in_specs=[pl.BlockSpec((1, _D, _UC), imap(k)) for k in range(4)],
        out_specs=pl.BlockSpec((_UC, 128), lambda f, u: (f * _NUB + u, 0)),
        out_shape=jax.ShapeDtypeStruct((_F * _RPF, 128), jnp.float32),
    )(tab_t, tab_t, tab_t, tab_t)


_SC_MESH = plsc.VectorSubcoreMesh(
    core_axis_name="c", subcore_axis_name="s",
    num_cores=_NC, num_subcores=_NS)


def _sc_prep_lin(didx2, lidx2, ltab):
    @functools.partial(
        pl.kernel,
        out_type=(
            jax.ShapeDtypeStruct((_NR, 128), jnp.float32),
            jax.ShapeDtypeStruct((_NR, 128), jnp.int32),
            jax.ShapeDtypeStruct((_NR, 128), jnp.int32),
        ),
        mesh=_SC_MESH,
        compiler_params=pltpu.CompilerParams(needs_layout_passes=False),
        scratch_types=[
            pltpu.VMEM((_RPW, 128), jnp.int32),
            pltpu.VMEM((_RPW, 128), jnp.int32),
            pltpu.VMEM((_RPW, 128), jnp.int32),
            pltpu.VMEM((_RPW, 128), jnp.float32),
            pltpu.SemaphoreType.DMA,
        ],
    )
    def body(didx_hbm, lidx_hbm, ltab_hbm, lin_out, m_out, q_out,
             m_v, q_v, lidx_v, lval_v, sem_l):
        wid = lax.axis_index("s") * _NC + lax.axis_index("c")
        row0 = wid * _RPW

        pltpu.sync_copy(didx_hbm.at[pl.ds(row0, _RPW)], m_v)
        pltpu.sync_copy(lidx_hbm.at[pl.ds(row0, _RPW)], lidx_v)

        iota = lax.iota(jnp.int32, 16)

        def prep(j, carry):
            for t in range(8):
                sl = pl.ds(t * 16, 16)
                f = ((j * 128 + t * 16) + iota) % _F
                raw = m_v[j, sl]
                m_v[j, sl] = f * _RPF + jax.lax.rem(raw, _RPF)
                q_v[j, sl] = jax.lax.div(raw, _RPF)
                lidx_v[j, sl] = lidx_v[j, sl] + f * _V
            return carry
        lax.fori_loop(0, _RPW, prep, 0)

        def lin_group(g, carry):
            for t in range(8):
                r = g * 8 + t
                pltpu.async_copy(ltab_hbm.at[lidx_v.at[r]], lval_v.at[r], sem_l)
            pltpu.make_async_copy(
                lin_out.at[pl.ds(row0, 8)],
                lval_v.at[pl.ds(g * 8, 8)], sem_l).wait()
            return carry
        lax.fori_loop(0, 13, lin_group, 0)
        pltpu.sync_copy(lval_v, lin_out.at[pl.ds(row0, _RPW)])
        pltpu.sync_copy(m_v, m_out.at[pl.ds(row0, _RPW)])
        pltpu.sync_copy(q_v, q_out.at[pl.ds(row0, _RPW)])

    return body(didx2, lidx2, ltab)


def _sc_gather(m2, q2, tw):
    @functools.partial(
        pl.kernel,
        out_type=jax.ShapeDtypeStruct((_FM_ROWS, 128), jnp.float32),
        mesh=_SC_MESH,
        compiler_params=pltpu.CompilerParams(needs_layout_passes=False),
        scratch_types=[
            pltpu.VMEM((_RPW, 128), jnp.int32),
            pltpu.VMEM((_RPW, 128), jnp.int32),
            pltpu.VMEM((128, 128), jnp.float32),
            pltpu.VMEM((128, 128), jnp.float32),
            pltpu.VMEM((128, 128), jnp.float32),
            pltpu.VMEM((128, 128), jnp.float32),
            pltpu.VMEM((32, 128), jnp.float32),
            pltpu.SemaphoreType.DMA,
            pltpu.SemaphoreType.DMA,
            pltpu.SemaphoreType.DMA,
            pltpu.SemaphoreType.DMA,
        ],
    )
    def body(m_hbm, q_hbm, tw_hbm, fm_out,
             m_v, q_v, w0, w1, w2, w3, ext, s0, s1, s2, s3):
        wid = lax.axis_index("s") * _NC + lax.axis_index("c")
        row0 = wid * _RPW
        wides = (w0, w1, w2, w3)
        sems = (s0, s1, s2, s3)

        pltpu.sync_copy(m_hbm.at[pl.ds(row0, _RPW)], m_v)
        pltpu.sync_copy(q_hbm.at[pl.ds(row0, _RPW)], q_v)

        iota = lax.iota(jnp.int32, 16)

        def extract(j, wide):
            jb = jnp.broadcast_to(j, (16,))

            def egrp(g0, carry):
                for u in range(8):
                    i = g0 * 8 + u
                    ib = jnp.broadcast_to(i, (16,))
                    col = plsc.load_gather(q_v, (jb, ib)) * 32 + iota
                    v0 = plsc.load_gather(wide, (ib, col))
                    v1 = plsc.load_gather(wide, (ib, col + 16))
                    ext[i // 4, pl.ds((i % 4) * 32, 16)] = v0
                    ext[i // 4, pl.ds((i % 4) * 32 + 16, 16)] = v1
                return carry
            lax.fori_loop(0, 16, egrp, 0)

        for p in range(3):
            pltpu.async_copy(tw_hbm.at[m_v.at[p]], wides[p], sems[p])

        def dnn_body(k, carry):
            for u in range(4):
                j = 4 * k + u
                pltpu.make_async_copy(
                    fm_out.at[pl.ds(0, 128)], wides[u], sems[u]).wait()

                @pl.when(j < _RPW - 3)
                def _():
                    pltpu.async_copy(
                        tw_hbm.at[m_v.at[j + 3]], wides[(u + 3) % 4],
                        sems[(u + 3) % 4])

                extract(j, wides[u])
                pltpu.sync_copy(ext, fm_out.at[pl.ds((row0 + j) * 32, 32)])
            return carry
        lax.fori_loop(0, _RPW // 4, dnn_body, 0)

    return body(m2, q2, tw)


def _tc_head(fm2, lval, dnn_dense, lin_dense,
             w1s, w1d, b1, g1, be1, w2, b2, g2, be2, w3, b3, lin_w, lin_b):
    bb = 512
    grid = (_B // bb,)

    def body(fm_ref, lv_ref, dd_ref, ld_ref,
             w1s_ref, w1d_ref, b1_ref, g1_ref, be1_ref,
             w2_ref, b2_ref, g2_ref, be2_ref,
             w3_ref, b3_ref, linw_ref, linb_ref, out_ref):
        fm = fm_ref[...]
        r = lax.broadcasted_iota(jnp.int32, (_F * _D, _D), 0)
        c = lax.broadcasted_iota(jnp.int32, (_F * _D, _D), 1)
        s = (r % _D == c).astype(jnp.float32)
        dn = (((1,), (1,)), ((), ()))
        mm = lambda x, w: lax.dot_general(
            x, w, dimension_numbers=dn, preferred_element_type=jnp.float32)
        sum_e = lax.dot_general(fm, s, dimension_numbers=(((1,), (0,)), ((), ())),
                                preferred_element_type=jnp.float32)
        ssq = lax.dot_general(fm * fm, s, dimension_numbers=(((1,), (0,)), ((), ())),
                              preferred_element_type=jnp.float32)
        cross = 0.5 * jnp.sum(sum_e * sum_e - ssq, axis=1, keepdims=True)

        h = mm(fm, w1s_ref[...]) + mm(dd_ref[...], w1d_ref[...]) + b1_ref[...]
        h = jnp.maximum(h, 0.0)
        m = jnp.mean(h, axis=1, keepdims=True)
        xc = h - m
        v = jnp.mean(xc * xc, axis=1, keepdims=True)
        h = xc * lax.rsqrt(v + 1e-5) * g1_ref[...] + be1_ref[...]

        h = jnp.maximum(mm(h, w2_ref[...]) + b2_ref[...], 0.0)
        m = jnp.mean(h, axis=1, keepdims=True)
        xc = h - m
        v = jnp.mean(xc * xc, axis=1, keepdims=True)
        h = xc * lax.rsqrt(v + 1e-5) * g2_ref[...] + be2_ref[...]

        dnn_logit = jnp.maximum(
            jnp.sum(h * w3_ref[...], axis=1, keepdims=True) + b3_ref[0, 0], 0.0)

        lin_logit = (jnp.sum(ld_ref[...] * linw_ref[...], axis=1, keepdims=True)
                     + linb_ref[0, 0]
                     + jnp.sum(lv_ref[...], axis=1, keepdims=True))
        out_ref[...] = lin_logit + dnn_logit + cross

    full = lambda shape: pl.BlockSpec(shape, lambda i: (0, 0))
    return pl.pallas_call(
        body,
        grid=grid,
        in_specs=[
            pl.BlockSpec((bb, _F * _D), lambda i: (i, 0)),
            pl.BlockSpec((bb, _F), lambda i: (i, 0)),
            pl.BlockSpec((bb, _DENSE), lambda i: (i, 0)),
            pl.BlockSpec((bb, _DENSE), lambda i: (i, 0)),
            full((128, _F * _D)), full((128, _DENSE)),
            full((1, 128)), full((1, 128)), full((1, 128)),
            full((64, 128)), full((1, 64)), full((1, 64)), full((1, 64)),
            full((1, 64)), full((1, 1)), full((1, _DENSE)), full((1, 1)),
        ],
        out_specs=pl.BlockSpec((bb, 1), lambda i: (i, 0)),
        out_shape=jax.ShapeDtypeStruct((_B, 1), jnp.float32),
    )(fm2, lval, dnn_dense, lin_dense,
      w1s, w1d, b1, g1, be1, w2, b2, g2, be2, w3, b3, lin_w, lin_b)


def kernel(linear_dense_data, dnn_dense_data, linear_tables, dnn_tables,
           lin_W, lin_b, W1, b1, ln1_g, ln1_b, W2, b2, ln2_g, ln2_b, W3, b3,
           linear_sparse_data, dnn_sparse_data):
    didx2 = dnn_sparse_data.astype(jnp.int32).reshape(_NR, 128)
    lidx2 = linear_sparse_data.astype(jnp.int32).reshape(_NR, 128)
    tab_t = jnp.transpose(dnn_tables, (0, 2, 1))
    tw = _tc_pack_table(tab_t)
    ltab = linear_tables.reshape(_F * _V)

    lin_rows, m2, q2 = _sc_prep_lin(didx2, lidx2, ltab)
    fm_rows = _sc_gather(m2, q2, tw)
    fm2 = fm_rows.reshape(_B, _F * _D)
    lval = lin_rows.reshape(_B, _F)

    w1d = W1[:, :_DENSE]
    w1s = W1[:, _DENSE:]
    out = _tc_head(
        fm2, lval, dnn_dense_data, linear_dense_data,
        w1s, w1d,
        b1.reshape(1, 128), ln1_g.reshape(1, 128), ln1_b.reshape(1, 128),
        W2, b2.reshape(1, 64), ln2_g.reshape(1, 64), ln2_b.reshape(1, 64),
        W3.reshape(1, 64), b3.reshape(1, 1),
        lin_W.reshape(1, _DENSE), lin_b.reshape(1, 1))
    return out


# speedup vs baseline: 11.9075x; 1.0701x over previous
import functools

import jax
import jax.numpy as jnp
from jax import lax
from jax.experimental import pallas as pl
from jax.experimental.pallas import tpu as pltpu
from jax.experimental.pallas import tpu_sc as plsc

_B = 16384
_F = 26
_V = 100000
_D = 32
_DENSE = 13

_NC = 2
_NS = 16
_NW = _NC * _NS

_NR = (_B * _F) // 128
_RPW = _NR // _NW

_RPF = 25088
_UC = 6272
_NUB = _RPF // _UC
_FM_ROWS = (_B * _F * _D) // 128


def _tc_pack_table(tab_t):
    def body(x0, x1, x2, x3, o_ref):
        o_ref[...] = jnp.concatenate(
            [x0[0], x1[0], x2[0], x3[0]], axis=0).T

    def imap(k):
        return lambda f, u: (f, 0, k * _NUB + u)

    return pl.pallas_call(
        body,
        grid=(_F, _NUB),
        in_specs=[pl.BlockSpec((1, _D, _UC), imap(k)) for k in range(4)],
        out_specs=pl.BlockSpec((_UC, 128), lambda f, u: (f * _NUB + u, 0)),
        out_shape=jax.ShapeDtypeStruct((_F * _RPF, 128), jnp.float32),
    )(tab_t, tab_t, tab_t, tab_t)


_SC_MESH = plsc.VectorSubcoreMesh(
    core_axis_name="c", subcore_axis_name="s",
    num_cores=_NC, num_subcores=_NS)


def _sc_prep_lin(didx2, lidx2, ltab):
    @functools.partial(
        pl.kernel,
        out_type=(
            jax.ShapeDtypeStruct((_NR, 128), jnp.float32),
            jax.ShapeDtypeStruct((_NR, 128), jnp.int32),
            jax.ShapeDtypeStruct((_NR, 128), jnp.int32),
        ),
        mesh=_SC_MESH,
        compiler_params=pltpu.CompilerParams(needs_layout_passes=False),
        scratch_types=[
            pltpu.VMEM((_RPW, 128), jnp.int32),
            pltpu.VMEM((_RPW, 128), jnp.int32),
            pltpu.VMEM((_RPW, 128), jnp.int32),
            pltpu.VMEM((_RPW, 128), jnp.float32),
            pltpu.SemaphoreType.DMA,
        ],
    )
    def body(didx_hbm, lidx_hbm, ltab_hbm, lin_out, m_out, q_out,
             m_v, q_v, lidx_v, lval_v, sem_l):
        wid = lax.axis_index("s") * _NC + lax.axis_index("c")
        row0 = wid * _RPW

        pltpu.sync_copy(didx_hbm.at[pl.ds(row0, _RPW)], m_v)
        pltpu.sync_copy(lidx_hbm.at[pl.ds(row0, _RPW)], lidx_v)

        iota = lax.iota(jnp.int32, 16)

        def prep(j, carry):
            for t in range(8):
                sl = pl.ds(t * 16, 16)
                f = ((j * 128 + t * 16) + iota) % _F
                raw = m_v[j, sl]
                m_v[j, sl] = f * _RPF + jax.lax.rem(raw, _RPF)
                q_v[j, sl] = jax.lax.div(raw, _RPF)
                lidx_v[j, sl] = lidx_v[j, sl] + f * _V
            return carry
        lax.fori_loop(0, _RPW, prep, 0)

        def lin_group(g, carry):
            for t in range(8):
                r = g * 8 + t
                pltpu.async_copy(ltab_hbm.at[lidx_v.at[r]], lval_v.at[r], sem_l)
            pltpu.make_async_copy(
                lin_out.at[pl.ds(row0, 8)],
                lval_v.at[pl.ds(g * 8, 8)], sem_l).wait()
            return carry
        lax.fori_loop(0, 13, lin_group, 0)
        pltpu.sync_copy(lval_v, lin_out.at[pl.ds(row0, _RPW)])
        pltpu.sync_copy(m_v, m_out.at[pl.ds(row0, _RPW)])
        pltpu.sync_copy(q_v, q_out.at[pl.ds(row0, _RPW)])

    return body(didx2, lidx2, ltab)


def _sc_gather(m2, q2, tw, h):
    hrw = _RPW // 2

    @functools.partial(
        pl.kernel,
        out_type=jax.ShapeDtypeStruct((_NR * 16, 128), jnp.float32),
        mesh=_SC_MESH,
        compiler_params=pltpu.CompilerParams(needs_layout_passes=False),
        scratch_types=[
            pltpu.VMEM((_RPW, 128), jnp.int32),
            pltpu.VMEM((_RPW, 128), jnp.int32),
            pltpu.VMEM((128, 128), jnp.float32),
            pltpu.VMEM((128, 128), jnp.float32),
            pltpu.VMEM((128, 128), jnp.float32),
            pltpu.VMEM((128, 128), jnp.float32),
            pltpu.VMEM((32, 128), jnp.float32),
            pltpu.SemaphoreType.DMA,
            pltpu.SemaphoreType.DMA,
            pltpu.SemaphoreType.DMA,
            pltpu.SemaphoreType.DMA,
        ],
    )
    def body(m_hbm, q_hbm, tw_hbm, fm_out,
             m_v, q_v, w0, w1, w2, w3, ext, s0, s1, s2, s3):
        wid = lax.axis_index("s") * _NC + lax.axis_index("c")
        row0 = wid * _RPW
        out0 = wid * hrw
        wides = (w0, w1, w2, w3)
        sems = (s0, s1, s2, s3)

        pltpu.sync_copy(m_hbm.at[pl.ds(row0, _RPW)], m_v)
        pltpu.sync_copy(q_hbm.at[pl.ds(row0, _RPW)], q_v)

        iota = lax.iota(jnp.int32, 16)

        def extract(j, wide):
            jb = jnp.broadcast_to(j, (16,))

            def egrp(g0, carry):
                for u in range(8):
                    i = g0 * 8 + u
                    ib = jnp.broadcast_to(i, (16,))
                    col = plsc.load_gather(q_v, (jb, ib)) * 32 + iota
                    v0 = plsc.load_gather(wide, (ib, col))
                    v1 = plsc.load_gather(wide, (ib, col + 16))
                    ext[i // 4, pl.ds((i % 4) * 32, 16)] = v0
                    ext[i // 4, pl.ds((i % 4) * 32 + 16, 16)] = v1
                return carry
            lax.fori_loop(0, 16, egrp, 0)

        for p in range(3):
            pltpu.async_copy(tw_hbm.at[m_v.at[h * hrw + p]], wides[p], sems[p])

        def dnn_body(k, carry):
            for u in range(4):
                j = 4 * k + u
                pltpu.make_async_copy(
                    fm_out.at[pl.ds(0, 128)], wides[u], sems[u]).wait()

                @pl.when(j < hrw - 3)
                def _():
                    pltpu.async_copy(
                        tw_hbm.at[m_v.at[h * hrw + j + 3]], wides[(u + 3) % 4],
                        sems[(u + 3) % 4])

                extract(h * hrw + j, wides[u])
                pltpu.sync_copy(ext, fm_out.at[pl.ds((out0 + j) * 32, 32)])
            return carry
        lax.fori_loop(0, hrw // 4, dnn_body, 0)

    return body(m2, q2, tw)


def _tc_head(fm2, lval, dnn_dense, lin_dense,
             w1s, w1d, b1, g1, be1, w2, b2, g2, be2, w3, b3, lin_w, lin_b):
    bb = 512
    nb = fm2.shape[0]
    grid = (nb // bb,)

    def body(fm_ref, lv_ref, dd_ref, ld_ref,
             w1s_ref, w1d_ref, b1_ref, g1_ref, be1_ref,
             w2_ref, b2_ref, g2_ref, be2_ref,
             w3_ref, b3_ref, linw_ref, linb_ref, out_ref):
        fm = fm_ref[...]
        r = lax.broadcasted_iota(jnp.int32, (_F * _D, _D), 0)
        c = lax.broadcasted_iota(jnp.int32, (_F * _D, _D), 1)
        s = (r % _D == c).astype(jnp.float32)
        dn = (((1,), (1,)), ((), ()))
        mm = lambda x, w: lax.dot_general(
            x, w, dimension_numbers=dn, preferred_element_type=jnp.float32)
        sum_e = lax.dot_general(fm, s, dimension_numbers=(((1,), (0,)), ((), ())),
                                preferred_element_type=jnp.float32)
        ssq = lax.dot_general(fm * fm, s, dimension_numbers=(((1,), (0,)), ((), ())),
                              preferred_element_type=jnp.float32)
        cross = 0.5 * jnp.sum(sum_e * sum_e - ssq, axis=1, keepdims=True)

        h = mm(fm, w1s_ref[...]) + mm(dd_ref[...], w1d_ref[...]) + b1_ref[...]
        h = jnp.maximum(h, 0.0)
        m = jnp.mean(h, axis=1, keepdims=True)
        xc = h - m
        v = jnp.mean(xc * xc, axis=1, keepdims=True)
        h = xc * lax.rsqrt(v + 1e-5) * g1_ref[...] + be1_ref[...]

        h = jnp.maximum(mm(h, w2_ref[...]) + b2_ref[...], 0.0)
        m = jnp.mean(h, axis=1, keepdims=True)
        xc = h - m
        v = jnp.mean(xc * xc, axis=1, keepdims=True)
        h = xc * lax.rsqrt(v + 1e-5) * g2_ref[...] + be2_ref[...]

        dnn_logit = jnp.maximum(
            jnp.sum(h * w3_ref[...], axis=1, keepdims=True) + b3_ref[0, 0], 0.0)

        lin_logit = (jnp.sum(ld_ref[...] * linw_ref[...], axis=1, keepdims=True)
                     + linb_ref[0, 0]
                     + jnp.sum(lv_ref[...], axis=1, keepdims=True))
        out_ref[...] = lin_logit + dnn_logit + cross

    full = lambda shape: pl.BlockSpec(shape, lambda i: (0, 0))
    return pl.pallas_call(
        body,
        grid=grid,
        in_specs=[
            pl.BlockSpec((bb, _F * _D), lambda i: (i, 0)),
            pl.BlockSpec((bb, _F), lambda i: (i, 0)),
            pl.BlockSpec((bb, _DENSE), lambda i: (i, 0)),
            pl.BlockSpec((bb, _DENSE), lambda i: (i, 0)),
            full((128, _F * _D)), full((128, _DENSE)),
            full((1, 128)), full((1, 128)), full((1, 128)),
            full((64, 128)), full((1, 64)), full((1, 64)), full((1, 64)),
            full((1, 64)), full((1, 1)), full((1, _DENSE)), full((1, 1)),
        ],
        out_specs=pl.BlockSpec((bb, 1), lambda i: (i, 0)),
        out_shape=jax.ShapeDtypeStruct((nb, 1), jnp.float32),
    )(fm2, lval, dnn_dense, lin_dense,
      w1s, w1d, b1, g1, be1, w2, b2, g2, be2, w3, b3, lin_w, lin_b)


def kernel(linear_dense_data, dnn_dense_data, linear_tables, dnn_tables,
           lin_W, lin_b, W1, b1, ln1_g, ln1_b, W2, b2, ln2_g, ln2_b, W3, b3,
           linear_sparse_data, dnn_sparse_data):
    didx2 = dnn_sparse_data.astype(jnp.int32).reshape(_NR, 128)
    lidx2 = linear_sparse_data.astype(jnp.int32).reshape(_NR, 128)
    tab_t = jnp.transpose(dnn_tables, (0, 2, 1))
    tw = _tc_pack_table(tab_t)
    ltab = linear_tables.reshape(_F * _V)

    lin_rows, m2, q2 = _sc_prep_lin(didx2, lidx2, ltab)
    lval = lin_rows.reshape(_B, _F)

    w1d = W1[:, :_DENSE]
    w1s = W1[:, _DENSE:]
    weights = (w1s, w1d,
               b1.reshape(1, 128), ln1_g.reshape(1, 128), ln1_b.reshape(1, 128),
               W2, b2.reshape(1, 64), ln2_g.reshape(1, 64), ln2_b.reshape(1, 64),
               W3.reshape(1, 64), b3.reshape(1, 1),
               lin_W.reshape(1, _DENSE), lin_b.reshape(1, 1))

    half_b = _B // 2
    bpw = _B // _NW // 2

    def pick(x, h):
        return x.reshape(_NW, 2, bpw, x.shape[1])[:, h].reshape(half_b,
                                                                x.shape[1])

    outs = []
    for h in range(2):
        fm_h = _sc_gather(m2, q2, tw, h)
        outs.append(_tc_head(
            fm_h.reshape(half_b, _F * _D), pick(lval, h),
            pick(dnn_dense_data, h), pick(linear_dense_data, h),
            *weights))
    return jnp.stack(
        [o.reshape(_NW, bpw) for o in outs], axis=1).reshape(_B, 1)
